# baseline (speedup 1.0000x reference)
import functools

import jax
import jax.numpy as jnp
from jax import lax
from jax.experimental import pallas as pl
from jax.experimental.pallas import tpu as pltpu


def _sigmoid(v):
    return 0.5 * jnp.tanh(0.5 * v) + 0.5


def _lstm_mlp_kernel(x_ref, wih_ref, whh_ref, bg_ref,
                     w1_ref, b1_ref, w2_ref, b2_ref,
                     out_ref, h_sc, c_sc, xg_sc, hs_sc,
                     *, hp, t_chunk, tb):
    d = x_ref.shape[-1]

    @pl.when(pl.program_id(1) == 0)
    def _():
        h_sc[...] = jnp.zeros_like(h_sc)
        c_sc[...] = jnp.zeros_like(c_sc)

    x_flat = x_ref[...].reshape(t_chunk * tb, d)
    xg = jnp.dot(x_flat, wih_ref[...],
                 preferred_element_type=jnp.float32) + bg_ref[...]
    xg_sc[...] = xg.reshape(t_chunk, tb, 4 * hp).astype(xg_sc.dtype)

    whh = whh_ref[...]

    def _step(t, carry):
        h_bf, c = carry
        xg_t = xg_sc[t]
        i_g = _sigmoid(xg_t[:, 0 * hp:1 * hp] + jnp.dot(
            h_bf, whh[:, 0 * hp:1 * hp], preferred_element_type=jnp.float32))
        f_g = _sigmoid(xg_t[:, 1 * hp:2 * hp] + jnp.dot(
            h_bf, whh[:, 1 * hp:2 * hp], preferred_element_type=jnp.float32))
        g_g = jnp.tanh(xg_t[:, 2 * hp:3 * hp] + jnp.dot(
            h_bf, whh[:, 2 * hp:3 * hp], preferred_element_type=jnp.float32))
        o_g = _sigmoid(xg_t[:, 3 * hp:4 * hp] + jnp.dot(
            h_bf, whh[:, 3 * hp:4 * hp], preferred_element_type=jnp.float32))
        c_new = f_g * c + i_g * g_g
        h_new = (o_g * jnp.tanh(c_new)).astype(jnp.bfloat16)
        hs_sc[t] = h_new
        return h_new, c_new

    h_fin, c_fin = lax.fori_loop(0, t_chunk, _step, (h_sc[...], c_sc[...]),
                                 unroll=t_chunk)
    h_sc[...] = h_fin
    c_sc[...] = c_fin

    hsb = hs_sc[...].reshape(t_chunk * tb, hp)
    z = jnp.dot(hsb, w1_ref[...], preferred_element_type=jnp.float32)
    z = jnp.maximum(z + b1_ref[...], 0.0)
    r = jnp.sum(z.reshape(t_chunk, tb, hp) * w2_ref[...], axis=-1)
    out_ref[...] = r.T + b2_ref[0, 0]


def kernel(x_btd, w_ih, w_hh, b_gates, w1, b1, w2, b2):
    B, T, D = x_btd.shape
    Hp = w_hh.shape[0]

    t_chunk = 128 if (T % 128 == 0) else T
    assert T % t_chunk == 0 and t_chunk % 8 == 0
    tb = B
    while tb * t_chunk * (4 * Hp + Hp + D) * 2 > 56 * 1024 * 1024 and tb % 16 == 0:
        tb //= 2
    assert B % tb == 0

    body = functools.partial(_lstm_mlp_kernel, hp=Hp, t_chunk=t_chunk, tb=tb)
    rep = lambda shape: pl.BlockSpec(shape, lambda b, c: (0,) * len(shape))

    x_tbd = jnp.transpose(x_btd, (1, 0, 2)).astype(jnp.bfloat16)

    out_bt = pl.pallas_call(
        body,
        out_shape=jax.ShapeDtypeStruct((B, T), jnp.float32),
        grid=(B // tb, T // t_chunk),
        in_specs=[
            pl.BlockSpec((t_chunk, tb, D), lambda b, c: (c, b, 0)),
            rep((D, 4 * Hp)),
            rep((Hp, 4 * Hp)),
            rep((1, 4 * Hp)),
            rep((Hp, Hp)),
            rep((1, Hp)),
            rep((1, Hp)),
            rep((1, 1)),
        ],
        out_specs=pl.BlockSpec((tb, t_chunk), lambda b, c: (b, c)),
        scratch_shapes=[
            pltpu.VMEM((tb, Hp), jnp.bfloat16),
            pltpu.VMEM((tb, Hp), jnp.float32),
            pltpu.VMEM((t_chunk, tb, 4 * Hp), jnp.bfloat16),
            pltpu.VMEM((t_chunk, tb, Hp), jnp.bfloat16),
        ],
        compiler_params=pltpu.CompilerParams(
            dimension_semantics=("parallel", "arbitrary"),
            vmem_limit_bytes=63 * 1024 * 1024,
        ),
    )(x_tbd, w_ih, w_hh, b_gates, w1, b1, w2, b2)

    return out_bt[..., None]


# speedup vs baseline: 1.1362x; 1.1362x over previous
import functools

import jax
import jax.numpy as jnp
from jax import lax
from jax.experimental import pallas as pl
from jax.experimental.pallas import tpu as pltpu


def _sigmoid(v):
    return 0.5 * jnp.tanh(0.5 * v) + 0.5


def _lstm_mlp_kernel(x_ref, wih_ref, whh_ref, bg_ref,
                     w1_ref, b1_ref, w2_ref, b2_ref,
                     out_ref, h_sc, c_sc, xg_sc, hs_sc,
                     *, hp, t_chunk, tb):
    d = x_ref.shape[-1]

    @pl.when(pl.program_id(1) == 0)
    def _():
        h_sc[...] = jnp.zeros_like(h_sc)
        c_sc[...] = jnp.zeros_like(c_sc)

    x_flat = x_ref[...].reshape(t_chunk * tb, d)
    xg = jnp.dot(x_flat, wih_ref[...],
                 preferred_element_type=jnp.float32) + bg_ref[...]
    xg_sc[...] = xg.reshape(t_chunk, tb, 4 * hp).astype(xg_sc.dtype)

    whh = whh_ref[...]

    def _step(t, carry):
        h_bf, c = carry
        xg_t = xg_sc[t]
        i_g = _sigmoid(xg_t[:, 0 * hp:1 * hp] + jnp.dot(
            h_bf, whh[:, 0 * hp:1 * hp], preferred_element_type=jnp.float32))
        f_g = _sigmoid(xg_t[:, 1 * hp:2 * hp] + jnp.dot(
            h_bf, whh[:, 1 * hp:2 * hp], preferred_element_type=jnp.float32))
        g_g = jnp.tanh(xg_t[:, 2 * hp:3 * hp] + jnp.dot(
            h_bf, whh[:, 2 * hp:3 * hp], preferred_element_type=jnp.float32))
        o_g = _sigmoid(xg_t[:, 3 * hp:4 * hp] + jnp.dot(
            h_bf, whh[:, 3 * hp:4 * hp], preferred_element_type=jnp.float32))
        c_new = f_g * c + i_g * g_g
        h_new = (o_g * jnp.tanh(c_new)).astype(jnp.bfloat16)
        hs_sc[t] = h_new
        return h_new, c_new

    h_fin, c_fin = lax.fori_loop(0, t_chunk, _step, (h_sc[...], c_sc[...]),
                                 unroll=64)
    h_sc[...] = h_fin
    c_sc[...] = c_fin

    hsb = hs_sc[...].reshape(t_chunk * tb, hp)
    z = jnp.dot(hsb, w1_ref[...], preferred_element_type=jnp.float32)
    z = jnp.maximum(z + b1_ref[...], 0.0)
    r = jnp.sum(z.reshape(t_chunk, tb, hp) * w2_ref[...], axis=-1)
    out_ref[...] = r.T + b2_ref[0, 0]


def kernel(x_btd, w_ih, w_hh, b_gates, w1, b1, w2, b2):
    B, T, D = x_btd.shape
    Hp = w_hh.shape[0]

    t_chunk = 128 if (T % 128 == 0) else T
    assert T % t_chunk == 0 and t_chunk % 8 == 0
    tb = B
    while tb * t_chunk * (4 * Hp + Hp + D) * 2 > 56 * 1024 * 1024 and tb % 16 == 0:
        tb //= 2
    assert B % tb == 0

    body = functools.partial(_lstm_mlp_kernel, hp=Hp, t_chunk=t_chunk, tb=tb)
    rep = lambda shape: pl.BlockSpec(shape, lambda b, c: (0,) * len(shape))

    x_tbd = jnp.transpose(x_btd, (1, 0, 2)).astype(jnp.bfloat16)

    out_bt = pl.pallas_call(
        body,
        out_shape=jax.ShapeDtypeStruct((B, T), jnp.float32),
        grid=(B // tb, T // t_chunk),
        in_specs=[
            pl.BlockSpec((t_chunk, tb, D), lambda b, c: (c, b, 0)),
            rep((D, 4 * Hp)),
            rep((Hp, 4 * Hp)),
            rep((1, 4 * Hp)),
            rep((Hp, Hp)),
            rep((1, Hp)),
            rep((1, Hp)),
            rep((1, 1)),
        ],
        out_specs=pl.BlockSpec((tb, t_chunk), lambda b, c: (b, c)),
        scratch_shapes=[
            pltpu.VMEM((tb, Hp), jnp.bfloat16),
            pltpu.VMEM((tb, Hp), jnp.float32),
            pltpu.VMEM((t_chunk, tb, 4 * Hp), jnp.bfloat16),
            pltpu.VMEM((t_chunk, tb, Hp), jnp.bfloat16),
        ],
        compiler_params=pltpu.CompilerParams(
            dimension_semantics=("parallel", "arbitrary"),
            vmem_limit_bytes=63 * 1024 * 1024,
        ),
    )(x_tbd, w_ih, w_hh, b_gates, w1, b1, w2, b2)

    return out_bt[..., None]
